# Initial kernel scaffold; baseline (speedup 1.0000x reference)
import jax
import jax.numpy as jnp
from jax.experimental import pallas as pl


def kernel(data, x, params):
    raise NotImplementedError("write your pallas kernel here")


# speedup vs baseline: 3.8458x; 3.8458x over previous
import jax
import jax.numpy as jnp
from jax import lax
from jax.experimental import pallas as pl

_K = 20


def _nm_block_body(data_ref, w1_ref, w2_ref, vrows_ref, vcols_ref, mats_ref,
                   out_ref):
    f32 = jnp.float32
    bf16 = jnp.bfloat16
    xr = data_ref[0]
    C, N = xr.shape
    Nf = f32(N)

    nd2r = jnp.sum(xr * xr, axis=0, keepdims=True)
    onr = xr / (jnp.sqrt(nd2r) + 1e-5)
    scol = jnp.sum(onr, axis=1, keepdims=True)
    Drow = (jnp.sum(onr * scol, axis=0, keepdims=True) - 1.0) / Nf
    drow = jax.nn.relu(jnp.tanh(Drow))

    xT = xr.T
    nd2c = jnp.sum(xT * xT, axis=1, keepdims=True)
    onc = xT / (jnp.sqrt(nd2c) + 1e-5)
    srow = jnp.sum(onc, axis=0, keepdims=True)
    Dcol = (jnp.sum(onc * srow, axis=1, keepdims=True) - 1.0) / Nf
    dcol = jax.nn.relu(jnp.tanh(Dcol))

    xk = jnp.concatenate([xr, drow], axis=0)
    xkT = jnp.concatenate([xT, dcol], axis=1)
    inner = -2.0 * lax.dot_general(xkT, xk, (((1,), (0,)), ((), ())),
                                   preferred_element_type=f32)
    xxrow = jnp.sum(xk * xk, axis=0, keepdims=True)
    xxcol = jnp.sum(xkT * xkT, axis=1, keepdims=True)
    S = ((-xxrow) - inner) - xxcol

    xkThi = xkT.astype(bf16)
    xkTlo = (xkT - xkThi.astype(f32)).astype(bf16)
    xkT_bf = xkT.astype(bf16)

    w1b = w1_ref[...].astype(bf16)
    w2b = w2_ref[...].astype(bf16)
    b0row = vrows_ref[0:1, :]
    g0row = vrows_ref[1:2, :]
    be0row = vrows_ref[2:3, :]
    sbn = jnp.sqrt(jnp.asarray(1.0 + 1e-5, f32))

    cx = lax.dot_general(xkT_bf, w1b, (((1,), (1,)), ((), ())),
                         preferred_element_type=f32)

    def argmin_onehot(cur):
        m = jnp.max(cur, axis=1, keepdims=True)
        iota = lax.broadcasted_iota(jnp.int32, (N, N), 1)
        sel = jnp.where(cur == m, iota, N)
        am = jnp.min(sel, axis=1, keepdims=True)
        return iota == am

    oh0 = argmin_onehot(S)
    S = jnp.where(oh0, -jnp.inf, S)
    hinit = jnp.zeros((N, 128), f32)

    def step(_, carry):
        cur, hmax = carry
        oh = argmin_onehot(cur)
        cur = jnp.where(oh, -jnp.inf, cur)
        ohb = oh.astype(bf16)
        fhi = lax.dot_general(ohb, xkThi, (((1,), (0,)), ((), ())),
                              preferred_element_type=f32)
        flo = lax.dot_general(ohb, xkTlo, (((1,), (0,)), ((), ())),
                              preferred_element_type=f32)
        dfn = (xkT - (fhi + flo)).astype(bf16)
        u = cx + lax.dot_general(dfn, w2b, (((1,), (1,)), ((), ())),
                                 preferred_element_type=f32)
        u = u + b0row
        u = jax.nn.relu((u / sbn) * g0row + be0row)
        return cur, jnp.maximum(hmax, u)

    _, hmax = lax.fori_loop(0, _K, step, (S, hinit))
    h = hmax.T

    for r in range(4):
        w1 = mats_ref[2 * r]
        w2 = mats_ref[2 * r + 1]
        ga = vcols_ref[:, 4 * r:4 * r + 1]
        ba = vcols_ref[:, 4 * r + 1:4 * r + 2]
        gb = vcols_ref[:, 4 * r + 2:4 * r + 3]
        bb = vcols_ref[:, 4 * r + 3:4 * r + 4]
        o = lax.dot_general(w1, h, (((1,), (0,)), ((), ())),
                            preferred_element_type=f32)
        o = o - jnp.mean(o, axis=1, keepdims=True)
        o = o / jnp.sqrt(jnp.mean(o * o, axis=1, keepdims=True) + 1e-5)
        o = jax.nn.relu((o / sbn) * ga + ba)
        o = lax.dot_general(w2, o, (((1,), (0,)), ((), ())),
                            preferred_element_type=f32)
        o = o - jnp.mean(o, axis=1, keepdims=True)
        o = o / jnp.sqrt(jnp.mean(o * o, axis=1, keepdims=True) + 1e-5)
        o = (o / sbn) * gb + bb
        h = jax.nn.relu(o + h)

    linw = vcols_ref[:, 16:17]
    linb = vcols_ref[0:1, 17:18]
    out_ref[0] = jnp.sum(h * linw, axis=0, keepdims=True) + linb


def kernel(data, x, params):
    del x
    B, C, N = data.shape
    f32 = jnp.float32
    W = params['conv0_w']
    W1 = W[:, :129]
    W2 = W[:, 129:]
    vrows = jnp.stack(
        [params['conv0_b'], params['bn0_g'], params['bn0_b']]).astype(f32)
    cols = []
    for r in range(1, 5):
        cols.append(params['res%d_g1' % r])
        cols.append(params['res%d_be1' % r])
        cols.append(params['res%d_g2' % r])
        cols.append(params['res%d_be2' % r])
    cols.append(params['lin_w'][0])
    cols.append(jnp.broadcast_to(params['lin_b'], (128,)))
    vcols = jnp.stack(cols, axis=1).astype(f32)
    mats = jnp.stack(
        [params['res%d_w%d' % (r, i)] for r in range(1, 5) for i in (1, 2)]
    ).astype(f32)

    out = pl.pallas_call(
        _nm_block_body,
        grid=(B,),
        in_specs=[
            pl.BlockSpec((1, C, N), lambda b: (b, 0, 0)),
            pl.BlockSpec((128, 129), lambda b: (0, 0)),
            pl.BlockSpec((128, 129), lambda b: (0, 0)),
            pl.BlockSpec((3, 128), lambda b: (0, 0)),
            pl.BlockSpec((128, 18), lambda b: (0, 0)),
            pl.BlockSpec((8, 128, 128), lambda b: (0, 0, 0)),
        ],
        out_specs=pl.BlockSpec((1, 1, N), lambda b: (b, 0, 0)),
        out_shape=jax.ShapeDtypeStruct((B, 1, N), f32),
    )(data, W1, W2, vrows, vcols, mats)
    return out.reshape(B, N)


# speedup vs baseline: 3.9650x; 1.0310x over previous
import jax
import jax.numpy as jnp
from jax import lax
from jax.experimental import pallas as pl

_K = 20


def _nm_block_body(data_ref, w1_ref, w2_ref, vrows_ref, vcols_ref, mats_ref,
                   out_ref):
    f32 = jnp.float32
    bf16 = jnp.bfloat16
    xr = data_ref[0]
    C, N = xr.shape
    Nf = f32(N)

    nd2r = jnp.sum(xr * xr, axis=0, keepdims=True)
    onr = xr / (jnp.sqrt(nd2r) + 1e-5)
    scol = jnp.sum(onr, axis=1, keepdims=True)
    Drow = (jnp.sum(onr * scol, axis=0, keepdims=True) - 1.0) / Nf
    drow = jax.nn.relu(jnp.tanh(Drow))

    xT = xr.T
    nd2c = jnp.sum(xT * xT, axis=1, keepdims=True)
    onc = xT / (jnp.sqrt(nd2c) + 1e-5)
    srow = jnp.sum(onc, axis=0, keepdims=True)
    Dcol = (jnp.sum(onc * srow, axis=1, keepdims=True) - 1.0) / Nf
    dcol = jax.nn.relu(jnp.tanh(Dcol))

    xk = jnp.concatenate([xr, drow], axis=0)
    xkT = jnp.concatenate([xT, dcol], axis=1)
    inner = -2.0 * lax.dot_general(xkT, xk, (((1,), (0,)), ((), ())),
                                   preferred_element_type=f32)
    xxrow = jnp.sum(xk * xk, axis=0, keepdims=True)
    xxcol = jnp.sum(xkT * xkT, axis=1, keepdims=True)
    S = ((-xxrow) - inner) - xxcol

    xkThi = xkT.astype(bf16)
    xkTlo = (xkT - xkThi.astype(f32)).astype(bf16)
    xkT_bf = xkT.astype(bf16)

    w1b = w1_ref[...].astype(bf16)
    w2b = w2_ref[...].astype(bf16)
    b0row = vrows_ref[0:1, :]
    g0row = vrows_ref[1:2, :]
    be0row = vrows_ref[2:3, :]
    sbn = jnp.sqrt(jnp.asarray(1.0 + 1e-5, f32))

    cx = lax.dot_general(xkT_bf, w1b, (((1,), (1,)), ((), ())),
                         preferred_element_type=f32)

    iota = lax.broadcasted_iota(jnp.int32, (N, N), 1)
    am0 = jnp.argmax(S, axis=1, keepdims=True).astype(jnp.int32)
    S = jnp.where(iota == am0, -jnp.inf, S)
    hinit = jnp.zeros((N, 128), f32)

    def step(t, carry):
        cur, hmax = carry
        am = jnp.argmax(cur, axis=1, keepdims=True).astype(jnp.int32)
        oh = iota == am
        cur = jnp.where(oh, -jnp.inf, cur)
        ohb = oh.astype(bf16)
        fhi = lax.dot_general(ohb, xkThi, (((1,), (0,)), ((), ())),
                              preferred_element_type=f32)
        flo = lax.dot_general(ohb, xkTlo, (((1,), (0,)), ((), ())),
                              preferred_element_type=f32)
        dfn = (xkT - (fhi + flo)).astype(bf16)
        u = cx + lax.dot_general(dfn, w2b, (((1,), (1,)), ((), ())),
                                 preferred_element_type=f32)
        u = u + b0row
        u = jax.nn.relu((u / sbn) * g0row + be0row)
        return cur, jnp.maximum(hmax, u)

    _, hmax = lax.fori_loop(0, _K, step, (S, hinit))
    h = hmax.T

    for r in range(4):
        w1 = mats_ref[2 * r]
        w2 = mats_ref[2 * r + 1]
        ga = vcols_ref[:, 4 * r:4 * r + 1]
        ba = vcols_ref[:, 4 * r + 1:4 * r + 2]
        gb = vcols_ref[:, 4 * r + 2:4 * r + 3]
        bb = vcols_ref[:, 4 * r + 3:4 * r + 4]
        o = lax.dot_general(w1, h, (((1,), (0,)), ((), ())),
                            preferred_element_type=f32)
        o = o - jnp.mean(o, axis=1, keepdims=True)
        o = o / jnp.sqrt(jnp.mean(o * o, axis=1, keepdims=True) + 1e-5)
        o = jax.nn.relu((o / sbn) * ga + ba)
        o = lax.dot_general(w2, o, (((1,), (0,)), ((), ())),
                            preferred_element_type=f32)
        o = o - jnp.mean(o, axis=1, keepdims=True)
        o = o / jnp.sqrt(jnp.mean(o * o, axis=1, keepdims=True) + 1e-5)
        o = (o / sbn) * gb + bb
        h = jax.nn.relu(o + h)

    linw = vcols_ref[:, 16:17]
    linb = vcols_ref[0:1, 17:18]
    out_ref[0] = jnp.sum(h * linw, axis=0, keepdims=True) + linb


def kernel(data, x, params):
    del x
    B, C, N = data.shape
    f32 = jnp.float32
    W = params['conv0_w']
    W1 = W[:, :129]
    W2 = W[:, 129:]
    vrows = jnp.stack(
        [params['conv0_b'], params['bn0_g'], params['bn0_b']]).astype(f32)
    cols = []
    for r in range(1, 5):
        cols.append(params['res%d_g1' % r])
        cols.append(params['res%d_be1' % r])
        cols.append(params['res%d_g2' % r])
        cols.append(params['res%d_be2' % r])
    cols.append(params['lin_w'][0])
    cols.append(jnp.broadcast_to(params['lin_b'], (128,)))
    vcols = jnp.stack(cols, axis=1).astype(f32)
    mats = jnp.stack(
        [params['res%d_w%d' % (r, i)] for r in range(1, 5) for i in (1, 2)]
    ).astype(f32)

    out = pl.pallas_call(
        _nm_block_body,
        grid=(B,),
        in_specs=[
            pl.BlockSpec((1, C, N), lambda b: (b, 0, 0)),
            pl.BlockSpec((128, 129), lambda b: (0, 0)),
            pl.BlockSpec((128, 129), lambda b: (0, 0)),
            pl.BlockSpec((3, 128), lambda b: (0, 0)),
            pl.BlockSpec((128, 18), lambda b: (0, 0)),
            pl.BlockSpec((8, 128, 128), lambda b: (0, 0, 0)),
        ],
        out_specs=pl.BlockSpec((1, 1, N), lambda b: (b, 0, 0)),
        out_shape=jax.ShapeDtypeStruct((B, 1, N), f32),
    )(data, W1, W2, vrows, vcols, mats)
    return out.reshape(B, N)


# speedup vs baseline: 3.9922x; 1.0068x over previous
import jax
import jax.numpy as jnp
from jax import lax
from jax.experimental import pallas as pl

_K = 20


def _nm_block_body(data_ref, w1_ref, w2_ref, vrows_ref, vcols_ref, mats_ref,
                   out_ref):
    f32 = jnp.float32
    bf16 = jnp.bfloat16
    xr = data_ref[0]
    C, N = xr.shape
    Nf = f32(N)

    nd2r = jnp.sum(xr * xr, axis=0, keepdims=True)
    onr = xr / (jnp.sqrt(nd2r) + 1e-5)
    scol = jnp.sum(onr, axis=1, keepdims=True)
    Drow = (jnp.sum(onr * scol, axis=0, keepdims=True) - 1.0) / Nf
    drow = jax.nn.relu(jnp.tanh(Drow))

    xT = xr.T
    nd2c = jnp.sum(xT * xT, axis=1, keepdims=True)
    onc = xT / (jnp.sqrt(nd2c) + 1e-5)
    srow = jnp.sum(onc, axis=0, keepdims=True)
    Dcol = (jnp.sum(onc * srow, axis=1, keepdims=True) - 1.0) / Nf
    dcol = jax.nn.relu(jnp.tanh(Dcol))

    xk = jnp.concatenate([xr, drow], axis=0)
    xkT = jnp.concatenate([xT, dcol], axis=1)
    inner = -2.0 * lax.dot_general(xkT, xk, (((1,), (0,)), ((), ())),
                                   preferred_element_type=f32)
    xxrow = jnp.sum(xk * xk, axis=0, keepdims=True)
    xxcol = jnp.sum(xkT * xkT, axis=1, keepdims=True)
    S = ((-xxrow) - inner) - xxcol

    xkThi = xkT.astype(bf16)
    xkTlo = (xkT - xkThi.astype(f32)).astype(bf16)
    xkT_bf = xkT.astype(bf16)

    w1b = w1_ref[...].astype(bf16)
    w2b = w2_ref[...].astype(bf16)
    b0row = vrows_ref[0:1, :]
    g0row = vrows_ref[1:2, :]
    be0row = vrows_ref[2:3, :]
    sbn = jnp.sqrt(jnp.asarray(1.0 + 1e-5, f32))

    cx = lax.dot_general(xkT_bf, w1b, (((1,), (1,)), ((), ())),
                         preferred_element_type=f32)

    iota = lax.broadcasted_iota(jnp.int32, (N, N), 1)
    am0 = jnp.argmax(S, axis=1, keepdims=True)
    cur = jnp.where(iota == am0, -jnp.inf, S)
    am1 = jnp.argmax(cur, axis=1, keepdims=True)
    cur = jnp.where(iota == am1, -jnp.inf, cur)
    hinit = jnp.zeros((N, 128), f32)

    def step(t, carry):
        cur, am, hmax = carry
        ohb = (iota == am).astype(bf16)
        fhi = lax.dot_general(ohb, xkThi, (((1,), (0,)), ((), ())),
                              preferred_element_type=f32)
        flo = lax.dot_general(ohb, xkTlo, (((1,), (0,)), ((), ())),
                              preferred_element_type=f32)
        dfn = (xkT - (fhi + flo)).astype(bf16)
        u = cx + lax.dot_general(dfn, w2b, (((1,), (1,)), ((), ())),
                                 preferred_element_type=f32)
        u = u + b0row
        u = jax.nn.relu((u / sbn) * g0row + be0row)
        hmax = jnp.maximum(hmax, u)
        amn = jnp.argmax(cur, axis=1, keepdims=True)
        cur = jnp.where(iota == amn, -jnp.inf, cur)
        return cur, amn, hmax

    _, _, hmax = lax.fori_loop(0, _K, step, (cur, am1, hinit))
    h = hmax.T

    for r in range(4):
        w1 = mats_ref[2 * r]
        w2 = mats_ref[2 * r + 1]
        ga = vcols_ref[:, 4 * r:4 * r + 1]
        ba = vcols_ref[:, 4 * r + 1:4 * r + 2]
        gb = vcols_ref[:, 4 * r + 2:4 * r + 3]
        bb = vcols_ref[:, 4 * r + 3:4 * r + 4]
        o = lax.dot_general(w1, h, (((1,), (0,)), ((), ())),
                            preferred_element_type=f32)
        o = o - jnp.mean(o, axis=1, keepdims=True)
        o = o / jnp.sqrt(jnp.mean(o * o, axis=1, keepdims=True) + 1e-5)
        o = jax.nn.relu((o / sbn) * ga + ba)
        o = lax.dot_general(w2, o, (((1,), (0,)), ((), ())),
                            preferred_element_type=f32)
        o = o - jnp.mean(o, axis=1, keepdims=True)
        o = o / jnp.sqrt(jnp.mean(o * o, axis=1, keepdims=True) + 1e-5)
        o = (o / sbn) * gb + bb
        h = jax.nn.relu(o + h)

    linw = vcols_ref[:, 16:17]
    linb = vcols_ref[0:1, 17:18]
    out_ref[0] = jnp.sum(h * linw, axis=0, keepdims=True) + linb


def kernel(data, x, params):
    del x
    B, C, N = data.shape
    f32 = jnp.float32
    W = params['conv0_w']
    W1 = W[:, :129]
    W2 = W[:, 129:]
    vrows = jnp.stack(
        [params['conv0_b'], params['bn0_g'], params['bn0_b']]).astype(f32)
    cols = []
    for r in range(1, 5):
        cols.append(params['res%d_g1' % r])
        cols.append(params['res%d_be1' % r])
        cols.append(params['res%d_g2' % r])
        cols.append(params['res%d_be2' % r])
    cols.append(params['lin_w'][0])
    cols.append(jnp.broadcast_to(params['lin_b'], (128,)))
    vcols = jnp.stack(cols, axis=1).astype(f32)
    mats = jnp.stack(
        [params['res%d_w%d' % (r, i)] for r in range(1, 5) for i in (1, 2)]
    ).astype(f32)

    out = pl.pallas_call(
        _nm_block_body,
        grid=(B,),
        in_specs=[
            pl.BlockSpec((1, C, N), lambda b: (b, 0, 0)),
            pl.BlockSpec((128, 129), lambda b: (0, 0)),
            pl.BlockSpec((128, 129), lambda b: (0, 0)),
            pl.BlockSpec((3, 128), lambda b: (0, 0)),
            pl.BlockSpec((128, 18), lambda b: (0, 0)),
            pl.BlockSpec((8, 128, 128), lambda b: (0, 0, 0)),
        ],
        out_specs=pl.BlockSpec((1, 1, N), lambda b: (b, 0, 0)),
        out_shape=jax.ShapeDtypeStruct((B, 1, N), f32),
    )(data, W1, W2, vrows, vcols, mats)
    return out.reshape(B, N)
